# baseline (speedup 1.0000x reference)
import functools

import jax
import jax.numpy as jnp
from jax import lax
from jax.experimental import pallas as pl
from jax.experimental.pallas import tpu as pltpu
from jax.experimental.pallas import tpu_sc as plsc

_B, _L, _NB, _LB, _D, _V = 1024, 200, 5, 50, 128, 100000
_NEG = _NB * _LB
_NEGP = 256
_LP = 208
_NC, _NS, _LANES = 2, 16, 16
_NW = _NC * _NS
_RPW = _B // _NW
_C = _D // _LANES
_NH0, _NH1 = 128, _NEG - 128


def _zeros8():
  return tuple(jnp.zeros((_LANES,), jnp.float32) for _ in range(_C))


def _sc_uae(tokens, negs, table, wt, bias):
  mesh = plsc.VectorSubcoreMesh(core_axis_name="c", subcore_axis_name="s")

  @functools.partial(
      pl.kernel,
      out_type=(
          jax.ShapeDtypeStruct((_B, _D), jnp.float32),
          jax.ShapeDtypeStruct((_B, _D), jnp.float32),
      ),
      mesh=mesh,
      compiler_params=pltpu.CompilerParams(needs_layout_passes=False),
      scratch_types=[
          pltpu.VMEM((_RPW, _L), jnp.int32),
          pltpu.VMEM((_RPW, _NEGP), jnp.int32),
          pltpu.VMEM((_L, _D), jnp.float32),
          pltpu.VMEM((_L, _D), jnp.float32),
          pltpu.VMEM((_NH0, _D), jnp.float32),
          pltpu.VMEM((_NH1, _D), jnp.float32),
          pltpu.VMEM((_D, _D // 2), jnp.int32),
          pltpu.VMEM((_D,), jnp.float32),
          pltpu.VMEM((_D,), jnp.float32),
          pltpu.VMEM((_LP,), jnp.float32),
          pltpu.VMEM((_LP,), jnp.float32),
          pltpu.VMEM((_D,), jnp.float32),
          pltpu.VMEM((_D,), jnp.float32),
          pltpu.SemaphoreType.DMA,
          pltpu.SemaphoreType.DMA,
      ],
  )
  def k(tok_hbm, neg_hbm, tab_hbm, wt_hbm, b_hbm, out_hbm, nout_hbm,
        tok_idx, neg_idx, ebuf0, ebuf1, nbuf0, nbuf1, wt_v, b_v, m_v,
        e_v, w_v, o_v, no_v, sem_t, sem_n):
    wid = lax.axis_index("s") * _NC + lax.axis_index("c")
    base = wid * _RPW
    ebufs = (ebuf0, ebuf1)

    pltpu.sync_copy(wt_hbm, wt_v)
    pltpu.sync_copy(b_hbm, b_v)
    pltpu.sync_copy(tok_hbm.at[pl.ds(base, _RPW)], tok_idx)
    pltpu.sync_copy(neg_hbm.at[pl.ds(base, _RPW)], neg_idx)

    def tok_gather(i, buf):
      pltpu.async_copy(tab_hbm.at[tok_idx.at[i, pl.ds(0, 128)]],
                       buf.at[pl.ds(0, 128)], sem_t)
      pltpu.async_copy(tab_hbm.at[tok_idx.at[i, pl.ds(128, _L - 128)]],
                       buf.at[pl.ds(128, _L - 128)], sem_t)

    def tok_wait(buf):
      pltpu.make_async_copy(tab_hbm.at[tok_idx.at[0, pl.ds(0, 128)]],
                            buf.at[pl.ds(0, 128)], sem_t).wait()
      pltpu.make_async_copy(tab_hbm.at[tok_idx.at[0, pl.ds(128, _L - 128)]],
                            buf.at[pl.ds(128, _L - 128)], sem_t).wait()

    lane = lax.iota(jnp.int32, _LANES)

    tok_gather(0, ebuf0)

    def row_body(i, erows, enext):
      cn0 = pltpu.async_copy(tab_hbm.at[neg_idx.at[i, pl.ds(0, _NH0)]],
                             nbuf0, sem_n)
      cn1 = pltpu.async_copy(tab_hbm.at[neg_idx.at[i, pl.ds(_NH0, _NH1)]],
                             nbuf1, sem_n)
      tok_wait(erows)
      inext = jnp.minimum(i + 1, _RPW - 1)
      tok_gather(inext, enext)

      def mean_body(lb, acc):
        for u in range(8):
          l = lb * 8 + u
          ec = [erows[l, pl.ds(c * _LANES, _LANES)] for c in range(_C)]
          acc = tuple(acc[c] + ec[c] for c in range(_C))
          for h in range(_C // 2):
            pk = plsc.pack(ec[2 * h], ec[2 * h + 1],
                           format=plsc.PackFormat.INTERLEAVED)
            erows[l, pl.ds(h * _LANES, _LANES)] = plsc.bitcast(
                pk, jnp.float32)
        return acc
      acc = lax.fori_loop(0, _L // 8, mean_body, _zeros8())
      for c in range(_C):
        m_v[pl.ds(c * _LANES, _LANES)] = acc[c] * (1.0 / _L)

      def mv_body(jc, y):
        m16 = m_v[pl.ds(jc * _LANES, _LANES)]
        for kk in range(_LANES):
          j = jc * _LANES + kk
          wr = []
          for h in range(_C // 2):
            wa, wb = plsc.unpack(
                plsc.bitcast(wt_v[j, pl.ds(h * _LANES, _LANES)],
                             jnp.bfloat16),
                format=plsc.PackFormat.INTERLEAVED)
            wr += [wa, wb]
          y = tuple(y[c] + m16[kk] * wr[c] for c in range(_C))
        return y
      y = lax.fori_loop(0, _D // _LANES, mv_body, _zeros8())
      y = tuple(y[c] + b_v[pl.ds(c * _LANES, _LANES)] for c in range(_C))

      def erow_bf(l):
        ec = []
        for h in range(_C // 2):
          ea, eb = plsc.unpack(
              plsc.bitcast(erows[l, pl.ds(h * _LANES, _LANES)],
                           jnp.bfloat16),
              format=plsc.PackFormat.INTERLEAVED)
          ec += [ea, eb]
        return ec

      def eij_chunk(lc, carry):
        e16 = jnp.zeros((_LANES,), jnp.float32)
        for kk in range(_LANES):
          l = lc * _LANES + kk
          ec = erow_bf(l)
          q = [ec[c] * y[c] for c in range(_C)]
          q = [q[0] + q[1], q[2] + q[3], q[4] + q[5], q[6] + q[7]]
          p = (q[0] + q[1]) + (q[2] + q[3])
          e16 = jnp.where(lane == kk, jnp.sum(p), e16)
        e_v[pl.ds(lc * _LANES, _LANES)] = e16
        return carry
      lax.fori_loop(0, _L // _LANES, eij_chunk, 0)
      e16 = jnp.zeros((_LANES,), jnp.float32)
      for kk in range(_L - (_L // _LANES) * _LANES):
        l = (_L // _LANES) * _LANES + kk
        ec = erow_bf(l)
        q = [ec[c] * y[c] for c in range(_C)]
        q = [q[0] + q[1], q[2] + q[3], q[4] + q[5], q[6] + q[7]]
        p = (q[0] + q[1]) + (q[2] + q[3])
        e16 = jnp.where(lane == kk, jnp.sum(p), e16)
      e_v[pl.ds((_L // _LANES) * _LANES, _LANES)] = e16

      sacc = jnp.zeros((_LANES,), jnp.float32)
      for c in range(_LP // _LANES):
        x = e_v[pl.ds(c * _LANES, _LANES)]
        e2x = jnp.exp(x * 2.0)
        t = 1.0 - 2.0 / (e2x + 1.0)
        p = jnp.exp(t)
        if (c + 1) * _LANES > _L:
          p = jnp.where(lane < _L - c * _LANES, p, jnp.zeros_like(p))
        w_v[pl.ds(c * _LANES, _LANES)] = p
        sacc = sacc + p
      s = sacc[0]
      for kk in range(1, _LANES):
        s = s + sacc[kk]
      rs = 1.0 / jnp.broadcast_to(s, (_LANES,))

      def ws_chunk(lc, acc):
        w16 = w_v[pl.ds(lc * _LANES, _LANES)]
        for kk in range(_LANES):
          l = lc * _LANES + kk
          ec = erow_bf(l)
          acc = tuple(acc[c] + w16[kk] * ec[c] for c in range(_C))
        return acc
      oacc = lax.fori_loop(0, _L // _LANES, ws_chunk, _zeros8())
      w16 = w_v[pl.ds((_L // _LANES) * _LANES, _LANES)]
      for kk in range(_L - (_L // _LANES) * _LANES):
        l = (_L // _LANES) * _LANES + kk
        ec = erow_bf(l)
        oacc = tuple(oacc[c] + w16[kk] * ec[c] for c in range(_C))
      for c in range(_C):
        o_v[pl.ds(c * _LANES, _LANES)] = oacc[c] * rs
      pltpu.sync_copy(o_v, out_hbm.at[base + i])

      cn0.wait()
      cn1.wait()
      def neg0_body(nb, acc):
        for u in range(8):
          n = nb * 8 + u
          acc = tuple(acc[c] + nbuf0[n, pl.ds(c * _LANES, _LANES)]
                      for c in range(_C))
        return acc
      nacc = lax.fori_loop(0, _NH0 // 8, neg0_body, _zeros8())
      def neg1_body(nb, acc):
        for u in range(8):
          n = nb * 8 + u
          acc = tuple(acc[c] + nbuf1[n, pl.ds(c * _LANES, _LANES)]
                      for c in range(_C))
        return acc
      nacc = lax.fori_loop(0, _NH1 // 8, neg1_body, nacc)
      for n in range((_NH1 // 8) * 8, _NH1):
        nacc = tuple(nacc[c] + nbuf1[n, pl.ds(c * _LANES, _LANES)]
                     for c in range(_C))
      for c in range(_C):
        no_v[pl.ds(c * _LANES, _LANES)] = nacc[c] * (1.0 / _NEG)
      pltpu.sync_copy(no_v, nout_hbm.at[base + i])

    def pair_body(p, carry):
      row_body(2 * p, ebuf0, ebuf1)
      row_body(2 * p + 1, ebuf1, ebuf0)
      return carry
    lax.fori_loop(0, _RPW // 2, pair_body, 0)

    tok_wait(ebuf0)

  return k(tokens, negs, table, wt, bias)


def kernel(tokens, sentence_embs, neg_bags, token_embedding, att_W, att_b):
  negs = jnp.pad(neg_bags.reshape(_B, _NEG).astype(jnp.int32),
                 ((0, 0), (0, _NEGP - _NEG)))
  wt = att_W.T.astype(jnp.bfloat16)
  wt = wt.reshape(_D, _C // 2, 2, _LANES).transpose(0, 1, 3, 2)
  wt = lax.bitcast_convert_type(wt.reshape(_D, _D // 2, 2), jnp.int32)
  out, nmean = _sc_uae(tokens.astype(jnp.int32), negs, token_embedding,
                       wt, att_b)
  return out, nmean, sentence_embs


# speedup vs baseline: 1.1905x; 1.1905x over previous
import functools

import jax
import jax.numpy as jnp
from jax import lax
from jax.experimental import pallas as pl
from jax.experimental.pallas import tpu as pltpu
from jax.experimental.pallas import tpu_sc as plsc

_B, _L, _NB, _LB, _D, _V = 1024, 200, 5, 50, 128, 100000
_NEG = _NB * _LB
_NEGP = 256
_LP = 208
_NC, _NS, _LANES = 2, 16, 16
_NW = _NC * _NS
_RPW = _B // _NW
_C = _D // _LANES
_NH0, _NH1 = 128, _NEG - 128


def _zeros8():
  return tuple(jnp.zeros((_LANES,), jnp.float32) for _ in range(_C))


def _sc_uae(tokens, negs, table, wt, bias):
  mesh = plsc.VectorSubcoreMesh(core_axis_name="c", subcore_axis_name="s")

  @functools.partial(
      pl.kernel,
      out_type=(
          jax.ShapeDtypeStruct((_B, _D), jnp.float32),
          jax.ShapeDtypeStruct((_B, _D), jnp.float32),
      ),
      mesh=mesh,
      compiler_params=pltpu.CompilerParams(needs_layout_passes=False),
      scratch_types=[
          pltpu.VMEM((_RPW, _L), jnp.int32),
          pltpu.VMEM((_RPW, _NEGP), jnp.int32),
          pltpu.VMEM((_LP, _D), jnp.float32),
          pltpu.VMEM((_LP, _D), jnp.float32),
          pltpu.VMEM((_NH0, _D), jnp.float32),
          pltpu.VMEM((_NH0, _D), jnp.float32),
          pltpu.VMEM((_NH0,), jnp.int32),
          pltpu.VMEM((_NH0,), jnp.int32),
          pltpu.VMEM_SHARED((_NS, _D), jnp.float32),
          pltpu.VMEM((1, _D), jnp.float32),
          pltpu.VMEM((1, _D), jnp.float32),
          pltpu.VMEM((_D, _D), jnp.float32),
          pltpu.VMEM((_D,), jnp.float32),
          pltpu.VMEM((_D,), jnp.float32),
          pltpu.VMEM((_LP,), jnp.float32),
          pltpu.VMEM((_LP,), jnp.float32),
          pltpu.VMEM((_D,), jnp.float32),
          pltpu.VMEM((_D,), jnp.float32),
          pltpu.SemaphoreType.DMA,
          pltpu.SemaphoreType.DMA,
          pltpu.SemaphoreType.DMA,
      ],
  )
  def k(tok_hbm, neg_hbm, tab_hbm, wt_hbm, b_hbm, out_hbm, nout_hbm,
        tok_idx, neg_idx, ebuf0, ebuf1, nbuf0, nbuf1, zidx0, zidx1, nacc2,
        zrow, arow, wt_v, b_v, m_v, e_v, w_v, o_v, no_v, sem_t, sem_n,
        sem_a):
    wid = lax.axis_index("s") * _NC + lax.axis_index("c")
    base = wid * _RPW
    ebufs = (ebuf0, ebuf1)

    pltpu.sync_copy(wt_hbm, wt_v)
    pltpu.sync_copy(b_hbm, b_v)
    pltpu.sync_copy(tok_hbm.at[pl.ds(base, _RPW)], tok_idx)
    pltpu.sync_copy(neg_hbm.at[pl.ds(base, _RPW)], neg_idx)

    def tok_gather(i, buf):
      pltpu.async_copy(tab_hbm.at[tok_idx.at[i, pl.ds(0, 128)]],
                       buf.at[pl.ds(0, 128)], sem_t)
      pltpu.async_copy(tab_hbm.at[tok_idx.at[i, pl.ds(128, _L - 128)]],
                       buf.at[pl.ds(128, _L - 128)], sem_t)

    def tok_wait(buf):
      pltpu.make_async_copy(tab_hbm.at[tok_idx.at[0, pl.ds(0, 128)]],
                            buf.at[pl.ds(0, 128)], sem_t).wait()
      pltpu.make_async_copy(tab_hbm.at[tok_idx.at[0, pl.ds(128, _L - 128)]],
                            buf.at[pl.ds(128, _L - 128)], sem_t).wait()

    lane = lax.iota(jnp.int32, _LANES)
    zero16f = jnp.zeros((_LANES,), jnp.float32)
    zero16i = jnp.zeros((_LANES,), jnp.int32)

    sid = lax.axis_index("s")
    for cc in range(_NH0 // _LANES):
      zidx0[pl.ds(cc * _LANES, _LANES)] = zero16i + sid
      zidx1[pl.ds(cc * _LANES, _LANES)] = zero16i + sid
    for c in range(_C):
      zrow[0, pl.ds(c * _LANES, _LANES)] = zero16f
    for n in range(_NH1, _NH0):
      for c in range(_C):
        nbuf1[n, pl.ds(c * _LANES, _LANES)] = zero16f
    pltpu.sync_copy(zrow, nacc2.at[pl.ds(sid, 1)])

    def neg_gather(i):
      pltpu.async_copy(tab_hbm.at[neg_idx.at[i, pl.ds(0, _NH0)]],
                       nbuf0, sem_n)
      pltpu.async_copy(tab_hbm.at[neg_idx.at[i, pl.ds(_NH0, _NH1)]],
                       nbuf1.at[pl.ds(0, _NH1)], sem_n)

    def neg_wait():
      pltpu.make_async_copy(tab_hbm.at[neg_idx.at[0, pl.ds(0, _NH0)]],
                            nbuf0, sem_n).wait()
      pltpu.make_async_copy(tab_hbm.at[neg_idx.at[0, pl.ds(_NH0, _NH1)]],
                            nbuf1.at[pl.ds(0, _NH1)], sem_n).wait()

    tok_gather(0, ebuf0)
    neg_gather(0)

    def row_body(i, erows, enext):
      neg_wait()
      sa0 = pltpu.async_copy(nbuf0, nacc2.at[zidx0], sem_a, add=True)
      sa1 = pltpu.async_copy(nbuf1, nacc2.at[zidx1], sem_a, add=True)
      tok_wait(erows)
      inext = jnp.minimum(i + 1, _RPW - 1)
      tok_gather(inext, enext)

      def mean_body(lb, acc):
        for u in range(8):
          l = lb * 8 + u
          acc = tuple(acc[c] + erows[l, pl.ds(c * _LANES, _LANES)]
                      for c in range(_C))
        return acc
      acc = lax.fori_loop(0, _L // 8, mean_body, _zeros8())
      for c in range(_C):
        m_v[pl.ds(c * _LANES, _LANES)] = acc[c] * (1.0 / _L)

      def mv_body(jc, y):
        m16 = m_v[pl.ds(jc * _LANES, _LANES)]
        for kk in range(_LANES):
          j = jc * _LANES + kk
          y = tuple(y[c] + m16[kk] * wt_v[j, pl.ds(c * _LANES, _LANES)]
                    for c in range(_C))
        return y
      y = lax.fori_loop(0, _D // _LANES, mv_body, _zeros8())
      y = tuple(y[c] + b_v[pl.ds(c * _LANES, _LANES)] for c in range(_C))

      def eij_chunk(lc, carry):
        e16 = jnp.zeros((_LANES,), jnp.float32)
        for kk in range(_LANES):
          l = lc * _LANES + kk
          p = erows[l, pl.ds(0, _LANES)] * y[0]
          for c in range(1, _C):
            p = p + erows[l, pl.ds(c * _LANES, _LANES)] * y[c]
          e16 = jnp.where(lane == kk, jnp.sum(p), e16)
        e_v[pl.ds(lc * _LANES, _LANES)] = e16
        return carry
      lax.fori_loop(0, _L // _LANES, eij_chunk, 0)
      e16 = jnp.zeros((_LANES,), jnp.float32)
      for kk in range(_L - (_L // _LANES) * _LANES):
        l = (_L // _LANES) * _LANES + kk
        p = erows[l, pl.ds(0, _LANES)] * y[0]
        for c in range(1, _C):
          p = p + erows[l, pl.ds(c * _LANES, _LANES)] * y[c]
        e16 = jnp.where(lane == kk, jnp.sum(p), e16)
      e_v[pl.ds((_L // _LANES) * _LANES, _LANES)] = e16

      sacc = jnp.zeros((_LANES,), jnp.float32)
      for c in range(_LP // _LANES):
        x = e_v[pl.ds(c * _LANES, _LANES)]
        e2x = jnp.exp(x * 2.0)
        t = 1.0 - 2.0 / (e2x + 1.0)
        p = jnp.exp(t)
        if (c + 1) * _LANES > _L:
          p = jnp.where(lane < _L - c * _LANES, p, jnp.zeros_like(p))
        w_v[pl.ds(c * _LANES, _LANES)] = p
        sacc = sacc + p
      s = sacc[0]
      for kk in range(1, _LANES):
        s = s + sacc[kk]
      rs = 1.0 / jnp.broadcast_to(s, (_LANES,))

      def ws_chunk(lc, acc):
        w16 = w_v[pl.ds(lc * _LANES, _LANES)]
        for kk in range(_LANES):
          l = lc * _LANES + kk
          acc = tuple(acc[c] + w16[kk] * erows[l, pl.ds(c * _LANES, _LANES)]
                      for c in range(_C))
        return acc
      oacc = lax.fori_loop(0, _L // _LANES, ws_chunk, _zeros8())
      w16 = w_v[pl.ds((_L // _LANES) * _LANES, _LANES)]
      for kk in range(_L - (_L // _LANES) * _LANES):
        l = (_L // _LANES) * _LANES + kk
        oacc = tuple(oacc[c] + w16[kk] * erows[l, pl.ds(c * _LANES, _LANES)]
                     for c in range(_C))
      for c in range(_C):
        o_v[pl.ds(c * _LANES, _LANES)] = oacc[c] * rs
      pltpu.sync_copy(o_v, out_hbm.at[base + i])

      sa0.wait()
      sa1.wait()
      pltpu.sync_copy(nacc2.at[pl.ds(sid, 1)], arow)
      pltpu.sync_copy(zrow, nacc2.at[pl.ds(sid, 1)])
      for c in range(_C):
        no_v[pl.ds(c * _LANES, _LANES)] = (
            arow[0, pl.ds(c * _LANES, _LANES)] * (1.0 / _NEG))
      pltpu.sync_copy(no_v, nout_hbm.at[base + i])
      neg_gather(jnp.minimum(i + 1, _RPW - 1))

    def pair_body(p, carry):
      row_body(2 * p, ebuf0, ebuf1)
      row_body(2 * p + 1, ebuf1, ebuf0)
      return carry
    lax.fori_loop(0, _RPW // 2, pair_body, 0)

    tok_wait(ebuf0)
    neg_wait()

  return k(tokens, negs, table, wt, bias)


def kernel(tokens, sentence_embs, neg_bags, token_embedding, att_W, att_b):
  negs = jnp.pad(neg_bags.reshape(_B, _NEG).astype(jnp.int32),
                 ((0, 0), (0, _NEGP - _NEG)))
  out, nmean = _sc_uae(tokens.astype(jnp.int32), negs, token_embedding,
                       att_W.T, att_b)
  return out, nmean, sentence_embs


# speedup vs baseline: 1.3192x; 1.1081x over previous
import functools

import jax
import jax.numpy as jnp
from jax import lax
from jax.experimental import pallas as pl
from jax.experimental.pallas import tpu as pltpu
from jax.experimental.pallas import tpu_sc as plsc

_B, _L, _NB, _LB, _D, _V = 1024, 200, 5, 50, 128, 100000
_NEG = _NB * _LB
_NEGP = 256
_LP = 208
_NC, _NS, _LANES = 2, 16, 16
_NW = _NC * _NS
_RPW = _B // _NW
_C = _D // _LANES
_NH0, _NH1 = 128, _NEG - 128


def _zeros8():
  return tuple(jnp.zeros((_LANES,), jnp.float32) for _ in range(_C))


def _sc_uae(tokens, negs, table, wt, bias):
  mesh = plsc.VectorSubcoreMesh(core_axis_name="c", subcore_axis_name="s")

  @functools.partial(
      pl.kernel,
      out_type=(
          jax.ShapeDtypeStruct((_B, _D), jnp.float32),
          jax.ShapeDtypeStruct((_B, _D), jnp.float32),
      ),
      mesh=mesh,
      compiler_params=pltpu.CompilerParams(needs_layout_passes=False),
      scratch_types=[
          pltpu.VMEM((_RPW, _L), jnp.int32),
          pltpu.VMEM((_RPW, _NEGP), jnp.int32),
          pltpu.VMEM((_LP, _D), jnp.float32),
          pltpu.VMEM((_LP, _D), jnp.float32),
          pltpu.VMEM((_NH0, _D), jnp.float32),
          pltpu.VMEM((_NH1, _D), jnp.float32),
          pltpu.VMEM((_D, _D), jnp.float32),
          pltpu.VMEM((_D,), jnp.float32),
          pltpu.VMEM((_D,), jnp.float32),
          pltpu.VMEM((_LP,), jnp.float32),
          pltpu.VMEM((_LP,), jnp.float32),
          pltpu.VMEM((_D,), jnp.float32),
          pltpu.VMEM((_D,), jnp.float32),
          pltpu.SemaphoreType.DMA,
          pltpu.SemaphoreType.DMA,
      ],
  )
  def k(tok_hbm, neg_hbm, tab_hbm, wt_hbm, b_hbm, out_hbm, nout_hbm,
        tok_idx, neg_idx, ebuf0, ebuf1, nbuf0, nbuf1, wt_v, b_v, m_v, e_v,
        w_v, o_v, no_v, sem_t, sem_n):
    wid = lax.axis_index("s") * _NC + lax.axis_index("c")
    base = wid * _RPW
    ebufs = (ebuf0, ebuf1)

    pltpu.sync_copy(wt_hbm, wt_v)
    pltpu.sync_copy(b_hbm, b_v)
    pltpu.sync_copy(tok_hbm.at[pl.ds(base, _RPW)], tok_idx)
    pltpu.sync_copy(neg_hbm.at[pl.ds(base, _RPW)], neg_idx)

    def tok_gather(i, buf):
      pltpu.async_copy(tab_hbm.at[tok_idx.at[i, pl.ds(0, 128)]],
                       buf.at[pl.ds(0, 128)], sem_t)
      pltpu.async_copy(tab_hbm.at[tok_idx.at[i, pl.ds(128, _L - 128)]],
                       buf.at[pl.ds(128, _L - 128)], sem_t)

    def tok_wait(buf):
      pltpu.make_async_copy(tab_hbm.at[tok_idx.at[0, pl.ds(0, 128)]],
                            buf.at[pl.ds(0, 128)], sem_t).wait()
      pltpu.make_async_copy(tab_hbm.at[tok_idx.at[0, pl.ds(128, _L - 128)]],
                            buf.at[pl.ds(128, _L - 128)], sem_t).wait()

    lane = lax.iota(jnp.int32, _LANES)

    tok_gather(0, ebuf0)

    def row_body(i, erows, enext):
      cn0 = pltpu.async_copy(tab_hbm.at[neg_idx.at[i, pl.ds(0, _NH0)]],
                             nbuf0, sem_n)
      cn1 = pltpu.async_copy(tab_hbm.at[neg_idx.at[i, pl.ds(_NH0, _NH1)]],
                             nbuf1, sem_n)
      tok_wait(erows)
      inext = jnp.minimum(i + 1, _RPW - 1)
      tok_gather(inext, enext)

      def mean_body(lb, acc):
        for u in range(8):
          l = lb * 8 + u
          acc = tuple(acc[c] + erows[l, pl.ds(c * _LANES, _LANES)]
                      for c in range(_C))
        return acc
      acc = lax.fori_loop(0, _L // 8, mean_body, _zeros8())
      for c in range(_C):
        m_v[pl.ds(c * _LANES, _LANES)] = acc[c] * (1.0 / _L)

      def mv_body(jc, y):
        m16 = m_v[pl.ds(jc * _LANES, _LANES)]
        for kk in range(_LANES):
          j = jc * _LANES + kk
          y = tuple(y[c] + m16[kk] * wt_v[j, pl.ds(c * _LANES, _LANES)]
                    for c in range(_C))
        return y
      y = lax.fori_loop(0, _D // _LANES, mv_body, _zeros8())
      y = tuple(y[c] + b_v[pl.ds(c * _LANES, _LANES)] for c in range(_C))

      def dot16(l):
        q = [erows[l, pl.ds(c * _LANES, _LANES)] * y[c] for c in range(_C)]
        q = [q[0] + q[1], q[2] + q[3], q[4] + q[5], q[6] + q[7]]
        return jnp.sum((q[0] + q[1]) + (q[2] + q[3]))

      def eij_chunk(lc, carry):
        ea = jnp.zeros((_LANES,), jnp.float32)
        eb = jnp.zeros((_LANES,), jnp.float32)
        for kk in range(_LANES // 2):
          l = lc * _LANES + kk
          ea = jnp.where(lane == kk, dot16(l), ea)
          eb = jnp.where(lane == kk + 8, dot16(l + 8), eb)
        e_v[pl.ds(lc * _LANES, _LANES)] = ea + eb
        return carry
      lax.fori_loop(0, _L // _LANES, eij_chunk, 0)
      ea = jnp.zeros((_LANES,), jnp.float32)
      eb = jnp.zeros((_LANES,), jnp.float32)
      for kk in range((_L - (_L // _LANES) * _LANES) // 2):
        l = (_L // _LANES) * _LANES + kk
        ea = jnp.where(lane == kk, dot16(l), ea)
        eb = jnp.where(lane == kk + 4, dot16(l + 4), eb)
      e_v[pl.ds((_L // _LANES) * _LANES, _LANES)] = ea + eb

      sacc = jnp.zeros((_LANES,), jnp.float32)
      for c in range(_LP // _LANES):
        x = e_v[pl.ds(c * _LANES, _LANES)]
        e2x = jnp.exp(x * 2.0)
        t = 1.0 - 2.0 / (e2x + 1.0)
        p = jnp.exp(t)
        if (c + 1) * _LANES > _L:
          p = jnp.where(lane < _L - c * _LANES, p, jnp.zeros_like(p))
        w_v[pl.ds(c * _LANES, _LANES)] = p
        sacc = sacc + p
      s = sacc[0]
      for kk in range(1, _LANES):
        s = s + sacc[kk]
      rs = 1.0 / jnp.broadcast_to(s, (_LANES,))

      def ws_chunk(lc, acc):
        w16 = w_v[pl.ds(lc * _LANES, _LANES)]
        for kk in range(_LANES):
          l = lc * _LANES + kk
          acc = tuple(acc[c] + w16[kk] * erows[l, pl.ds(c * _LANES, _LANES)]
                      for c in range(_C))
        return acc
      oacc = lax.fori_loop(0, _L // _LANES, ws_chunk, _zeros8())
      w16 = w_v[pl.ds((_L // _LANES) * _LANES, _LANES)]
      for kk in range(_L - (_L // _LANES) * _LANES):
        l = (_L // _LANES) * _LANES + kk
        oacc = tuple(oacc[c] + w16[kk] * erows[l, pl.ds(c * _LANES, _LANES)]
                     for c in range(_C))
      for c in range(_C):
        o_v[pl.ds(c * _LANES, _LANES)] = oacc[c] * rs
      pltpu.sync_copy(o_v, out_hbm.at[base + i])

      cn0.wait()
      cn1.wait()
      def neg0_body(nb, acc):
        for u in range(8):
          n = nb * 8 + u
          acc = tuple(acc[c] + nbuf0[n, pl.ds(c * _LANES, _LANES)]
                      for c in range(_C))
        return acc
      nacc = lax.fori_loop(0, _NH0 // 8, neg0_body, _zeros8())
      def neg1_body(nb, acc):
        for u in range(8):
          n = nb * 8 + u
          acc = tuple(acc[c] + nbuf1[n, pl.ds(c * _LANES, _LANES)]
                      for c in range(_C))
        return acc
      nacc = lax.fori_loop(0, _NH1 // 8, neg1_body, nacc)
      for n in range((_NH1 // 8) * 8, _NH1):
        nacc = tuple(nacc[c] + nbuf1[n, pl.ds(c * _LANES, _LANES)]
                     for c in range(_C))
      for c in range(_C):
        no_v[pl.ds(c * _LANES, _LANES)] = nacc[c] * (1.0 / _NEG)
      pltpu.sync_copy(no_v, nout_hbm.at[base + i])

    def pair_body(p, carry):
      row_body(2 * p, ebuf0, ebuf1)
      row_body(2 * p + 1, ebuf1, ebuf0)
      return carry
    lax.fori_loop(0, _RPW // 2, pair_body, 0)

    tok_wait(ebuf0)

  return k(tokens, negs, table, wt, bias)


def kernel(tokens, sentence_embs, neg_bags, token_embedding, att_W, att_b):
  negs = jnp.pad(neg_bags.reshape(_B, _NEG).astype(jnp.int32),
                 ((0, 0), (0, _NEGP - _NEG)))
  out, nmean = _sc_uae(tokens.astype(jnp.int32), negs, token_embedding,
                       att_W.T, att_b)
  return out, nmean, sentence_embs


# speedup vs baseline: 1.3508x; 1.0239x over previous
import functools

import jax
import jax.numpy as jnp
from jax import lax
from jax.experimental import pallas as pl
from jax.experimental.pallas import tpu as pltpu
from jax.experimental.pallas import tpu_sc as plsc

_B, _L, _NB, _LB, _D, _V = 1024, 200, 5, 50, 128, 100000
_NEG = _NB * _LB
_NEGP = 256
_LP = 208
_NC, _NS, _LANES = 2, 16, 16
_NW = _NC * _NS
_RPW = _B // _NW
_C = _D // _LANES
_NH0, _NH1 = 128, _NEG - 128


def _zeros8():
  return tuple(jnp.zeros((_LANES,), jnp.float32) for _ in range(_C))


def _sc_uae(tokens, negs, table, wt, bias):
  mesh = plsc.VectorSubcoreMesh(core_axis_name="c", subcore_axis_name="s")

  @functools.partial(
      pl.kernel,
      out_type=(
          jax.ShapeDtypeStruct((_B, _D), jnp.float32),
          jax.ShapeDtypeStruct((_B, _D), jnp.float32),
      ),
      mesh=mesh,
      compiler_params=pltpu.CompilerParams(needs_layout_passes=False),
      scratch_types=[
          pltpu.VMEM((_RPW, _L), jnp.int32),
          pltpu.VMEM((_RPW, _NEGP), jnp.int32),
          pltpu.VMEM((_LP, _D), jnp.float32),
          pltpu.VMEM((_LP, _D), jnp.float32),
          pltpu.VMEM((_NH0, _D), jnp.float32),
          pltpu.VMEM((_NH1, _D), jnp.float32),
          pltpu.VMEM((_D, _D), jnp.float32),
          pltpu.VMEM((_D,), jnp.float32),
          pltpu.VMEM((_D,), jnp.float32),
          pltpu.VMEM((_LP,), jnp.float32),
          pltpu.VMEM((_LP,), jnp.float32),
          pltpu.VMEM((_D,), jnp.float32),
          pltpu.VMEM((_D,), jnp.float32),
          pltpu.SemaphoreType.DMA,
          pltpu.SemaphoreType.DMA,
          pltpu.SemaphoreType.DMA,
          pltpu.SemaphoreType.DMA,
      ],
  )
  def k(tok_hbm, neg_hbm, tab_hbm, wt_hbm, b_hbm, out_hbm, nout_hbm,
        tok_idx, neg_idx, ebuf0, ebuf1, nbuf0, nbuf1, wt_v, b_v, m_v, e_v,
        w_v, o_v, no_v, sem_t, sem_n, sem_o, sem_no):
    wid = lax.axis_index("s") * _NC + lax.axis_index("c")
    base = wid * _RPW
    ebufs = (ebuf0, ebuf1)

    pltpu.sync_copy(wt_hbm, wt_v)
    pltpu.sync_copy(b_hbm, b_v)
    pltpu.sync_copy(tok_hbm.at[pl.ds(base, _RPW)], tok_idx)
    pltpu.sync_copy(neg_hbm.at[pl.ds(base, _RPW)], neg_idx)

    def tok_gather(i, buf):
      pltpu.async_copy(tab_hbm.at[tok_idx.at[i, pl.ds(0, 128)]],
                       buf.at[pl.ds(0, 128)], sem_t)
      pltpu.async_copy(tab_hbm.at[tok_idx.at[i, pl.ds(128, _L - 128)]],
                       buf.at[pl.ds(128, _L - 128)], sem_t)

    def tok_wait(buf):
      pltpu.make_async_copy(tab_hbm.at[tok_idx.at[0, pl.ds(0, 128)]],
                            buf.at[pl.ds(0, 128)], sem_t).wait()
      pltpu.make_async_copy(tab_hbm.at[tok_idx.at[0, pl.ds(128, _L - 128)]],
                            buf.at[pl.ds(128, _L - 128)], sem_t).wait()

    lane = lax.iota(jnp.int32, _LANES)

    tok_gather(0, ebuf0)

    def row_body(i, erows, enext):
      tok_wait(erows)
      inext = jnp.minimum(i + 1, _RPW - 1)
      tok_gather(inext, enext)
      cn0 = pltpu.async_copy(tab_hbm.at[neg_idx.at[i, pl.ds(0, _NH0)]],
                             nbuf0, sem_n)
      cn1 = pltpu.async_copy(tab_hbm.at[neg_idx.at[i, pl.ds(_NH0, _NH1)]],
                             nbuf1, sem_n)

      def mean_body(lb, acc):
        for u in range(8):
          l = lb * 8 + u
          acc = tuple(acc[c] + erows[l, pl.ds(c * _LANES, _LANES)]
                      for c in range(_C))
        return acc
      acc = lax.fori_loop(0, _L // 8, mean_body, _zeros8())
      for c in range(_C):
        m_v[pl.ds(c * _LANES, _LANES)] = acc[c] * (1.0 / _L)

      def mv_body(jc, y):
        m16 = m_v[pl.ds(jc * _LANES, _LANES)]
        for kk in range(_LANES):
          j = jc * _LANES + kk
          y = tuple(y[c] + m16[kk] * wt_v[j, pl.ds(c * _LANES, _LANES)]
                    for c in range(_C))
        return y
      y = lax.fori_loop(0, _D // _LANES, mv_body, _zeros8())
      y = tuple(y[c] + b_v[pl.ds(c * _LANES, _LANES)] for c in range(_C))

      def dot16(l):
        q = [erows[l, pl.ds(c * _LANES, _LANES)] * y[c] for c in range(_C)]
        q = [q[0] + q[1], q[2] + q[3], q[4] + q[5], q[6] + q[7]]
        return jnp.sum((q[0] + q[1]) + (q[2] + q[3]))

      def eij_chunk(lc, carry):
        ea = jnp.zeros((_LANES,), jnp.float32)
        eb = jnp.zeros((_LANES,), jnp.float32)
        for kk in range(_LANES // 2):
          l = lc * _LANES + kk
          ea = jnp.where(lane == kk, dot16(l), ea)
          eb = jnp.where(lane == kk + 8, dot16(l + 8), eb)
        e_v[pl.ds(lc * _LANES, _LANES)] = ea + eb
        return carry
      lax.fori_loop(0, _L // _LANES, eij_chunk, 0)
      ea = jnp.zeros((_LANES,), jnp.float32)
      eb = jnp.zeros((_LANES,), jnp.float32)
      for kk in range((_L - (_L // _LANES) * _LANES) // 2):
        l = (_L // _LANES) * _LANES + kk
        ea = jnp.where(lane == kk, dot16(l), ea)
        eb = jnp.where(lane == kk + 4, dot16(l + 4), eb)
      e_v[pl.ds((_L // _LANES) * _LANES, _LANES)] = ea + eb

      sacc = jnp.zeros((_LANES,), jnp.float32)
      for c in range(_LP // _LANES):
        x = e_v[pl.ds(c * _LANES, _LANES)]
        e2x = jnp.exp(x * 2.0)
        t = 1.0 - 2.0 / (e2x + 1.0)
        p = jnp.exp(t)
        if (c + 1) * _LANES > _L:
          p = jnp.where(lane < _L - c * _LANES, p, jnp.zeros_like(p))
        w_v[pl.ds(c * _LANES, _LANES)] = p
        sacc = sacc + p
      s = sacc[0]
      for kk in range(1, _LANES):
        s = s + sacc[kk]
      rs = 1.0 / jnp.broadcast_to(s, (_LANES,))

      def ws_chunk(lc, acc):
        w16 = w_v[pl.ds(lc * _LANES, _LANES)]
        for kk in range(_LANES):
          l = lc * _LANES + kk
          acc = tuple(acc[c] + w16[kk] * erows[l, pl.ds(c * _LANES, _LANES)]
                      for c in range(_C))
        return acc
      oacc = lax.fori_loop(0, _L // _LANES, ws_chunk, _zeros8())
      w16 = w_v[pl.ds((_L // _LANES) * _LANES, _LANES)]
      for kk in range(_L - (_L // _LANES) * _LANES):
        l = (_L // _LANES) * _LANES + kk
        oacc = tuple(oacc[c] + w16[kk] * erows[l, pl.ds(c * _LANES, _LANES)]
                     for c in range(_C))
      @pl.when(i > 0)
      def _wait_prev_out():
        pltpu.make_async_copy(o_v, out_hbm.at[base], sem_o).wait()
      for c in range(_C):
        o_v[pl.ds(c * _LANES, _LANES)] = oacc[c] * rs
      pltpu.async_copy(o_v, out_hbm.at[base + i], sem_o)

      cn0.wait()
      cn1.wait()
      def neg0_body(nb, acc):
        for u in range(8):
          n = nb * 8 + u
          acc = tuple(acc[c] + nbuf0[n, pl.ds(c * _LANES, _LANES)]
                      for c in range(_C))
        return acc
      nacc = lax.fori_loop(0, _NH0 // 8, neg0_body, _zeros8())
      def neg1_body(nb, acc):
        for u in range(8):
          n = nb * 8 + u
          acc = tuple(acc[c] + nbuf1[n, pl.ds(c * _LANES, _LANES)]
                      for c in range(_C))
        return acc
      nacc = lax.fori_loop(0, _NH1 // 8, neg1_body, nacc)
      for n in range((_NH1 // 8) * 8, _NH1):
        nacc = tuple(nacc[c] + nbuf1[n, pl.ds(c * _LANES, _LANES)]
                     for c in range(_C))
      @pl.when(i > 0)
      def _wait_prev_nout():
        pltpu.make_async_copy(no_v, nout_hbm.at[base], sem_no).wait()
      for c in range(_C):
        no_v[pl.ds(c * _LANES, _LANES)] = nacc[c] * (1.0 / _NEG)
      pltpu.async_copy(no_v, nout_hbm.at[base + i], sem_no)

    def pair_body(p, carry):
      row_body(2 * p, ebuf0, ebuf1)
      row_body(2 * p + 1, ebuf1, ebuf0)
      return carry
    lax.fori_loop(0, _RPW // 2, pair_body, 0)

    tok_wait(ebuf0)
    pltpu.make_async_copy(o_v, out_hbm.at[base], sem_o).wait()
    pltpu.make_async_copy(no_v, nout_hbm.at[base], sem_no).wait()

  return k(tokens, negs, table, wt, bias)


def kernel(tokens, sentence_embs, neg_bags, token_embedding, att_W, att_b):
  negs = jnp.pad(neg_bags.reshape(_B, _NEG).astype(jnp.int32),
                 ((0, 0), (0, _NEGP - _NEG)))
  out, nmean = _sc_uae(tokens.astype(jnp.int32), negs, token_embedding,
                       att_W.T, att_b)
  return out, nmean, sentence_embs


# speedup vs baseline: 1.3685x; 1.0131x over previous
import functools

import jax
import jax.numpy as jnp
from jax import lax
from jax.experimental import pallas as pl
from jax.experimental.pallas import tpu as pltpu
from jax.experimental.pallas import tpu_sc as plsc

_B, _L, _NB, _LB, _D, _V = 1024, 200, 5, 50, 128, 100000
_NEG = _NB * _LB
_NEGP = 256
_LP = 208
_NC, _NS, _LANES = 2, 16, 16
_NW = _NC * _NS
_RPW = _B // _NW
_C = _D // _LANES
_NH0, _NH1 = 128, _NEG - 128


def _zeros8():
  return tuple(jnp.zeros((_LANES,), jnp.float32) for _ in range(_C))


def _sc_uae(tokens, negs, table, wt, bias):
  mesh = plsc.VectorSubcoreMesh(core_axis_name="c", subcore_axis_name="s")

  @functools.partial(
      pl.kernel,
      out_type=(
          jax.ShapeDtypeStruct((_B, _D), jnp.float32),
          jax.ShapeDtypeStruct((_B, _D), jnp.float32),
      ),
      mesh=mesh,
      compiler_params=pltpu.CompilerParams(needs_layout_passes=False),
      scratch_types=[
          pltpu.VMEM((_RPW, _L), jnp.int32),
          pltpu.VMEM((_RPW, _NEGP), jnp.int32),
          pltpu.VMEM((_LP, _D), jnp.float32),
          pltpu.VMEM((_LP, _D), jnp.float32),
          pltpu.VMEM((_NH0, _D), jnp.float32),
          pltpu.VMEM((_NH1, _D), jnp.float32),
          pltpu.VMEM((_D, _D), jnp.float32),
          pltpu.VMEM((_D,), jnp.float32),
          pltpu.VMEM((_D,), jnp.float32),
          pltpu.VMEM((_LP,), jnp.float32),
          pltpu.VMEM((_LP,), jnp.float32),
          pltpu.VMEM((_D,), jnp.float32),
          pltpu.VMEM((_D,), jnp.float32),
          pltpu.SemaphoreType.DMA,
          pltpu.SemaphoreType.DMA,
          pltpu.SemaphoreType.DMA,
          pltpu.SemaphoreType.DMA,
      ],
  )
  def k(tok_hbm, neg_hbm, tab_hbm, wt_hbm, b_hbm, out_hbm, nout_hbm,
        tok_idx, neg_idx, ebuf0, ebuf1, nbuf0, nbuf1, wt_v, b_v, m_v, e_v,
        w_v, o_v, no_v, sem_t, sem_n, sem_o, sem_no):
    wid = lax.axis_index("s") * _NC + lax.axis_index("c")
    base = wid * _RPW
    ebufs = (ebuf0, ebuf1)

    pltpu.sync_copy(wt_hbm, wt_v)
    pltpu.sync_copy(b_hbm, b_v)
    pltpu.sync_copy(tok_hbm.at[pl.ds(base, _RPW)], tok_idx)
    pltpu.sync_copy(neg_hbm.at[pl.ds(base, _RPW)], neg_idx)

    def tok_gather(i, buf):
      pltpu.async_copy(tab_hbm.at[tok_idx.at[i, pl.ds(0, 128)]],
                       buf.at[pl.ds(0, 128)], sem_t)
      pltpu.async_copy(tab_hbm.at[tok_idx.at[i, pl.ds(128, _L - 128)]],
                       buf.at[pl.ds(128, _L - 128)], sem_t)

    def tok_wait(buf):
      pltpu.make_async_copy(tab_hbm.at[tok_idx.at[0, pl.ds(0, 128)]],
                            buf.at[pl.ds(0, 128)], sem_t).wait()
      pltpu.make_async_copy(tab_hbm.at[tok_idx.at[0, pl.ds(128, _L - 128)]],
                            buf.at[pl.ds(128, _L - 128)], sem_t).wait()

    lane = lax.iota(jnp.int32, _LANES)

    tok_gather(0, ebuf0)

    def row_body(i, erows, enext):
      tok_wait(erows)
      inext = jnp.minimum(i + 1, _RPW - 1)
      tok_gather(inext, enext)
      cn0 = pltpu.async_copy(tab_hbm.at[neg_idx.at[i, pl.ds(0, _NH0)]],
                             nbuf0, sem_n)
      cn1 = pltpu.async_copy(tab_hbm.at[neg_idx.at[i, pl.ds(_NH0, _NH1)]],
                             nbuf1, sem_n)

      def mean_body(lb, acc):
        for u in range(8):
          l = lb * 8 + u
          acc = tuple(acc[c] + erows[l, pl.ds(c * _LANES, _LANES)]
                      for c in range(_C))
        return acc
      acc = lax.fori_loop(0, _L // 8, mean_body, _zeros8())
      for c in range(_C):
        m_v[pl.ds(c * _LANES, _LANES)] = acc[c] * (1.0 / _L)

      def mv_body(jc, y):
        m16 = m_v[pl.ds(jc * _LANES, _LANES)]
        for kk in range(_LANES):
          j = jc * _LANES + kk
          y = tuple(y[c] + m16[kk] * wt_v[j, pl.ds(c * _LANES, _LANES)]
                    for c in range(_C))
        return y
      y = lax.fori_loop(0, _D // _LANES, mv_body, _zeros8())
      y = tuple(y[c] + b_v[pl.ds(c * _LANES, _LANES)] for c in range(_C))

      def dot16(l):
        q = [erows[l, pl.ds(c * _LANES, _LANES)] * y[c] for c in range(_C)]
        q = [q[0] + q[1], q[2] + q[3], q[4] + q[5], q[6] + q[7]]
        return jnp.sum((q[0] + q[1]) + (q[2] + q[3]))

      def softw(x):
        e2x = jnp.exp(x * 2.0)
        return jnp.exp(1.0 - 2.0 / (e2x + 1.0))

      def eij_chunk(lc, sacc):
        ea = jnp.zeros((_LANES,), jnp.float32)
        eb = jnp.zeros((_LANES,), jnp.float32)
        for kk in range(_LANES // 2):
          l = lc * _LANES + kk
          ea = jnp.where(lane == kk, dot16(l), ea)
          eb = jnp.where(lane == kk + 8, dot16(l + 8), eb)
        pw = softw(ea + eb)
        w_v[pl.ds(lc * _LANES, _LANES)] = pw
        return sacc + pw
      sacc = lax.fori_loop(0, _L // _LANES, eij_chunk,
                           jnp.zeros((_LANES,), jnp.float32))
      ea = jnp.zeros((_LANES,), jnp.float32)
      eb = jnp.zeros((_LANES,), jnp.float32)
      for kk in range((_L - (_L // _LANES) * _LANES) // 2):
        l = (_L // _LANES) * _LANES + kk
        ea = jnp.where(lane == kk, dot16(l), ea)
        eb = jnp.where(lane == kk + 4, dot16(l + 4), eb)
      pw = jnp.where(lane < _L - (_L // _LANES) * _LANES,
                     softw(ea + eb), jnp.zeros((_LANES,), jnp.float32))
      w_v[pl.ds((_L // _LANES) * _LANES, _LANES)] = pw
      sacc = sacc + pw
      s = sacc[0]
      for kk in range(1, _LANES):
        s = s + sacc[kk]
      rs = 1.0 / jnp.broadcast_to(s, (_LANES,))

      def ws_chunk(lc, acc):
        w16 = w_v[pl.ds(lc * _LANES, _LANES)]
        for kk in range(_LANES):
          l = lc * _LANES + kk
          acc = tuple(acc[c] + w16[kk] * erows[l, pl.ds(c * _LANES, _LANES)]
                      for c in range(_C))
        return acc
      oacc = lax.fori_loop(0, _L // _LANES, ws_chunk, _zeros8())
      w16 = w_v[pl.ds((_L // _LANES) * _LANES, _LANES)]
      for kk in range(_L - (_L // _LANES) * _LANES):
        l = (_L // _LANES) * _LANES + kk
        oacc = tuple(oacc[c] + w16[kk] * erows[l, pl.ds(c * _LANES, _LANES)]
                     for c in range(_C))
      @pl.when(i > 0)
      def _wait_prev_out():
        pltpu.make_async_copy(o_v, out_hbm.at[base], sem_o).wait()
      for c in range(_C):
        o_v[pl.ds(c * _LANES, _LANES)] = oacc[c] * rs
      pltpu.async_copy(o_v, out_hbm.at[base + i], sem_o)

      cn0.wait()
      cn1.wait()
      def neg0_body(nb, acc):
        for u in range(8):
          n = nb * 8 + u
          acc = tuple(acc[c] + nbuf0[n, pl.ds(c * _LANES, _LANES)]
                      for c in range(_C))
        return acc
      nacc = lax.fori_loop(0, _NH0 // 8, neg0_body, _zeros8())
      def neg1_body(nb, acc):
        for u in range(8):
          n = nb * 8 + u
          acc = tuple(acc[c] + nbuf1[n, pl.ds(c * _LANES, _LANES)]
                      for c in range(_C))
        return acc
      nacc = lax.fori_loop(0, _NH1 // 8, neg1_body, nacc)
      for n in range((_NH1 // 8) * 8, _NH1):
        nacc = tuple(nacc[c] + nbuf1[n, pl.ds(c * _LANES, _LANES)]
                     for c in range(_C))
      @pl.when(i > 0)
      def _wait_prev_nout():
        pltpu.make_async_copy(no_v, nout_hbm.at[base], sem_no).wait()
      for c in range(_C):
        no_v[pl.ds(c * _LANES, _LANES)] = nacc[c] * (1.0 / _NEG)
      pltpu.async_copy(no_v, nout_hbm.at[base + i], sem_no)

    def pair_body(p, carry):
      row_body(2 * p, ebuf0, ebuf1)
      row_body(2 * p + 1, ebuf1, ebuf0)
      return carry
    lax.fori_loop(0, _RPW // 2, pair_body, 0)

    tok_wait(ebuf0)
    pltpu.make_async_copy(o_v, out_hbm.at[base], sem_o).wait()
    pltpu.make_async_copy(no_v, nout_hbm.at[base], sem_no).wait()

  return k(tokens, negs, table, wt, bias)


def kernel(tokens, sentence_embs, neg_bags, token_embedding, att_W, att_b):
  negs = jnp.pad(neg_bags.reshape(_B, _NEG).astype(jnp.int32),
                 ((0, 0), (0, _NEGP - _NEG)))
  out, nmean = _sc_uae(tokens.astype(jnp.int32), negs, token_embedding,
                       att_W.T, att_b)
  return out, nmean, sentence_embs
